# Initial kernel scaffold; baseline (speedup 1.0000x reference)
import jax
import jax.numpy as jnp
from jax.experimental import pallas as pl


def kernel(pos, edge_index, batch, params):
    raise NotImplementedError("write your pallas kernel here")


# speedup vs baseline: 1.7682x; 1.7682x over previous
import jax
import jax.numpy as jnp
from jax import lax
from jax.experimental import pallas as pl

P = 100
K = 20
F32 = jnp.float32


def _iota_j():
    return lax.broadcasted_iota(jnp.int32, (P, P), 1)


def _mm(a, w):
    return jnp.dot(a, w, preferred_element_type=F32)


def _gmm(a, w):
    return jnp.dot(a, w, preferred_element_type=F32,
                   precision=lax.Precision.HIGHEST)


def _bz(t):
    return t.astype(jnp.bfloat16).astype(F32)


def _knn_cov_body(pos_ref, idx_ref, cov_ref):
    pos = pos_ref[0]
    dot = jnp.dot(pos, pos.T, preferred_element_type=F32)
    ii = lax.broadcasted_iota(jnp.int32, (P, P), 0)
    jj = _iota_j()
    eye = (ii == jj).astype(F32)
    sq_col = jnp.sum(pos * pos, axis=1, keepdims=True)
    sq_row = jnp.sum(eye * sq_col, axis=0, keepdims=True)
    d = (sq_col + sq_row) - 2.0 * dot + eye * 1e10

    idxm = jnp.zeros((P, K), jnp.int32)
    kk = lax.broadcasted_iota(jnp.int32, (P, K), 1)
    cab = [[None] * 3 for _ in range(3)]
    for k in range(K):
        m = jnp.min(d, axis=1, keepdims=True)
        amin = jnp.min(jnp.where(d == m, jj, P), axis=1, keepdims=True)
        oh = (jj == amin)
        idxm = jnp.where(kk == k, amin, idxm)
        ps = _gmm(oh.astype(F32), pos)
        r = [ps[:, c:c + 1] - pos[:, c:c + 1] for c in range(3)]
        for a in range(3):
            for b in range(a, 3):
                t = r[a] * r[b]
                cab[a][b] = t if cab[a][b] is None else cab[a][b] + t
        d = jnp.where(oh, 1e30, d)

    idx_ref[0] = idxm
    c9 = lax.broadcasted_iota(jnp.int32, (P, 9), 1)
    cov = jnp.zeros((P, 9), F32)
    for a in range(3):
        for b in range(3):
            v = cab[a][b] if b >= a else cab[b][a]
            cov = jnp.where(c9 == 3 * a + b, v / K, cov)
    cov_ref[0] = cov


def _rl_cols(rlm, k):
    return [rlm[:, 3 * k + c:3 * k + c + 1] for c in range(3)]


def _z1(rl, W1a, b1a):
    z = (_bz(rl[0]) * _bz(W1a[0:1, :]) + _bz(rl[1]) * _bz(W1a[1:2, :])
         + _bz(rl[2]) * _bz(W1a[2:3, :])) + b1a
    return jnp.maximum(z, 0.0)


def _stats1_body(rl_ref, W1a_ref, b1a_ref, s_ref, q_ref):
    rlm = rl_ref[0]
    W1a, b1a = W1a_ref[...], b1a_ref[...]
    s = jnp.zeros((1, 64), F32)
    q = jnp.zeros((1, 64), F32)
    for k in range(K):
        a1 = _z1(_rl_cols(rlm, k), W1a, b1a)
        s = s + jnp.sum(a1, axis=0, keepdims=True)
        q = q + jnp.sum(a1 * a1, axis=0, keepdims=True)
    s_ref[0] = s
    q_ref[0] = q


def _stats2_body(rl_ref, W1a_ref, b1a_ref, sc1_ref, sh1_ref, W1b_ref, b1b_ref,
                 s_ref, q_ref):
    rlm = rl_ref[0]
    W1a, b1a = W1a_ref[...], b1a_ref[...]
    sc1, sh1 = sc1_ref[...], sh1_ref[...]
    W1b, b1b = W1b_ref[...], b1b_ref[...]
    s = jnp.zeros((1, 64), F32)
    q = jnp.zeros((1, 64), F32)
    for k in range(K):
        a1 = _z1(_rl_cols(rlm, k), W1a, b1a)
        a2 = jnp.maximum(_mm(a1 * sc1 + sh1, W1b) + b1b, 0.0)
        s = s + jnp.sum(a2, axis=0, keepdims=True)
        q = q + jnp.sum(a2 * a2, axis=0, keepdims=True)
    s_ref[0] = s
    q_ref[0] = q


def _stats3_body(rl_ref, W1a_ref, b1a_ref, sc1_ref, sh1_ref, W1b_ref, b1b_ref,
                 sc2_ref, sh2_ref, W1c_ref, b1c_ref, s_ref, q_ref):
    rlm = rl_ref[0]
    W1a, b1a = W1a_ref[...], b1a_ref[...]
    sc1, sh1 = sc1_ref[...], sh1_ref[...]
    W1b, b1b = W1b_ref[...], b1b_ref[...]
    sc2, sh2 = sc2_ref[...], sh2_ref[...]
    W1c, b1c = W1c_ref[...], b1c_ref[...]
    s = jnp.zeros((1, 64), F32)
    q = jnp.zeros((1, 64), F32)
    for k in range(K):
        a1 = _z1(_rl_cols(rlm, k), W1a, b1a)
        a2 = jnp.maximum(_mm(a1 * sc1 + sh1, W1b) + b1b, 0.0)
        a3 = jnp.maximum(_mm(a2 * sc2 + sh2, W1c) + b1c, 0.0)
        s = s + jnp.sum(a3, axis=0, keepdims=True)
        q = q + jnp.sum(a3 * a3, axis=0, keepdims=True)
    s_ref[0] = s
    q_ref[0] = q


def _stage2(rl, oh, feat1, W20, W21, W22, b2):
    fs = _gmm(oh, feat1)
    return (_mm(fs * rl[0], W20) + _mm(fs * rl[1], W21)
            + _mm(fs * rl[2], W22)) + b2


def _feat1_stage2_body(rl_ref, idx_ref, W1a_ref, b1a_ref, sc1_ref, sh1_ref,
                       W1b_ref, b1b_ref, sc2_ref, sh2_ref, W1c_ref, b1c_ref,
                       sc3_ref, sh3_ref, W20_ref, W21_ref, W22_ref, b2_ref,
                       feat1_ref, s_ref, q_ref):
    rlm, idxm = rl_ref[0], idx_ref[0]
    W1a, b1a = W1a_ref[...], b1a_ref[...]
    sc1, sh1 = sc1_ref[...], sh1_ref[...]
    W1b, b1b = W1b_ref[...], b1b_ref[...]
    sc2, sh2 = sc2_ref[...], sh2_ref[...]
    W1c, b1c = W1c_ref[...], b1c_ref[...]
    sc3, sh3 = sc3_ref[...], sh3_ref[...]
    W20, W21, W22, b2 = W20_ref[...], W21_ref[...], W22_ref[...], b2_ref[...]

    feat1 = None
    for k in range(K):
        a1 = _z1(_rl_cols(rlm, k), W1a, b1a)
        a2 = jnp.maximum(_mm(a1 * sc1 + sh1, W1b) + b1b, 0.0)
        a3 = jnp.maximum(_mm(a2 * sc2 + sh2, W1c) + b1c, 0.0)
        h = a3 * sc3 + sh3
        feat1 = h if feat1 is None else jnp.maximum(feat1, h)
    feat1_ref[0] = feat1

    jj = _iota_j()
    s = jnp.zeros((1, 128), F32)
    q = jnp.zeros((1, 128), F32)
    for k in range(K):
        oh = (jj == idxm[:, k:k + 1]).astype(F32)
        a = jnp.maximum(
            _stage2(_rl_cols(rlm, k), oh, feat1, W20, W21, W22, b2), 0.0)
        s = s + jnp.sum(a, axis=0, keepdims=True)
        q = q + jnp.sum(a * a, axis=0, keepdims=True)
    s_ref[0] = s
    q_ref[0] = q


def _feat2_head1_body(rl_ref, idx_ref, feat1_ref, W20_ref, W21_ref, W22_ref,
                      b2_ref, sc4_ref, sh4_ref, Wn1_ref, bn1_ref, y_ref):
    rlm, idxm = rl_ref[0], idx_ref[0]
    feat1 = feat1_ref[0]
    W20, W21, W22, b2 = W20_ref[...], W21_ref[...], W22_ref[...], b2_ref[...]
    sc4, sh4 = sc4_ref[...], sh4_ref[...]
    Wn1, bn1 = Wn1_ref[...], bn1_ref[...]

    jj = _iota_j()
    feat2 = None
    for k in range(K):
        oh = (jj == idxm[:, k:k + 1]).astype(F32)
        z2 = _stage2(_rl_cols(rlm, k), oh, feat1, W20, W21, W22, b2)
        h2 = jnp.maximum(z2, 0.0) * sc4 + sh4
        feat2 = h2 if feat2 is None else jnp.maximum(feat2, h2)
    y1 = _mm(feat2, Wn1) + bn1
    y_ref[0] = jnp.max(y1, axis=0, keepdims=True)


def _head_body(y_ref, Wn2_ref, bn2_ref, g1_ref, be1_ref, g2_ref, be2_ref,
               Wn3_ref, bn3_ref, g3_ref, be3_ref, Wn4_ref, bn4_ref, out_ref):
    def bn(x, g, be):
        m = jnp.mean(x, axis=0, keepdims=True)
        v = jnp.mean((x - m) * (x - m), axis=0, keepdims=True)
        return (x - m) / jnp.sqrt(v + 1e-5) * g + be

    y = jnp.maximum(y_ref[...], 0.0)
    y = bn(y, g1_ref[...], be1_ref[...])
    y = jnp.maximum(_mm(y, Wn2_ref[...]) + bn2_ref[...], 0.0)
    y = bn(y, g2_ref[...], be2_ref[...])
    y = jnp.maximum(_mm(y, Wn3_ref[...]) + bn3_ref[...], 0.0)
    y = bn(y, g3_ref[...], be3_ref[...])
    z = _mm(y, Wn4_ref[...]) + bn4_ref[...]
    zm = jnp.max(z, axis=1, keepdims=True)
    u = z - zm
    out_ref[...] = u - jnp.log(jnp.sum(jnp.exp(u), axis=1, keepdims=True))


def _cloud_spec(t):
    return pl.BlockSpec((1,) + t, lambda i: (i, 0, 0))


def _bcast_spec(shape):
    n = len(shape)
    return pl.BlockSpec(shape, lambda i: (0,) * n)


def kernel(pos, edge_index, batch, params):
    del edge_index, batch
    p = params
    N = pos.shape[0]
    B = N // P
    E = N * K
    pos = pos.astype(F32)
    pos3 = pos.reshape(B, P, 3)

    idx3, cov3 = pl.pallas_call(
        _knn_cov_body,
        grid=(B,),
        in_specs=[_cloud_spec((P, 3))],
        out_specs=[_cloud_spec((P, K)), _cloud_spec((P, 9))],
        out_shape=[jax.ShapeDtypeStruct((B, P, K), jnp.int32),
                   jax.ShapeDtypeStruct((B, P, 9), F32)],
    )(pos3)

    cov = cov3.reshape(N, 3, 3)
    _, V = jnp.linalg.eigh(cov)
    V_t = jnp.swapaxes(V, -1, -2)
    src = (idx3 + (jnp.arange(B) * P)[:, None, None]).reshape(-1)
    dst = jnp.repeat(jnp.arange(N), K)
    rel = pos[src] - pos[dst]
    rl = jnp.einsum('eij,ej->ei', V_t[dst], rel)
    rl3 = rl.reshape(B, P, K * 3)

    def stats(s, q, n, g, be):
        m = jnp.sum(s, axis=(0, 1)) / n
        v = jnp.sum(q, axis=(0, 1)) / n - m * m
        sc = (g / jnp.sqrt(v + 1e-5))[None, :]
        sh = (be - m * sc[0])[None, :]
        return sc.astype(F32), sh.astype(F32)

    W1a = p['W1a'].astype(F32)
    b1a = p['b1a'].astype(F32)[None, :]
    W1b = p['W1b'].astype(F32)
    b1b = p['b1b'].astype(F32)[None, :]
    W1c = p['W1c'].astype(F32)
    b1c = p['b1c'].astype(F32)[None, :]
    rl_in = _cloud_spec((P, K * 3))
    stat_outs = [_cloud_spec((1, 64)), _cloud_spec((1, 64))]
    stat_shapes = [jax.ShapeDtypeStruct((B, 1, 64), F32)] * 2
    v64 = _bcast_spec((1, 64))

    s1, q1 = pl.pallas_call(
        _stats1_body, grid=(B,),
        in_specs=[rl_in, _bcast_spec((3, 64)), v64],
        out_specs=stat_outs, out_shape=stat_shapes,
    )(rl3, W1a, b1a)
    sc1, sh1 = stats(s1, q1, E, p['g1a'], p['be1a'])

    s2, q2 = pl.pallas_call(
        _stats2_body, grid=(B,),
        in_specs=[rl_in, _bcast_spec((3, 64)), v64, v64, v64,
                  _bcast_spec((64, 64)), v64],
        out_specs=stat_outs, out_shape=stat_shapes,
    )(rl3, W1a, b1a, sc1, sh1, W1b, b1b)
    sc2, sh2 = stats(s2, q2, E, p['g1b'], p['be1b'])

    s3, q3 = pl.pallas_call(
        _stats3_body, grid=(B,),
        in_specs=[rl_in, _bcast_spec((3, 64)), v64, v64, v64,
                  _bcast_spec((64, 64)), v64, v64, v64,
                  _bcast_spec((64, 64)), v64],
        out_specs=stat_outs, out_shape=stat_shapes,
    )(rl3, W1a, b1a, sc1, sh1, W1b, b1b, sc2, sh2, W1c, b1c)
    sc3, sh3 = stats(s3, q3, E, p['g1c'], p['be1c'])

    W20 = p['W2'][0::3].astype(F32)
    W21 = p['W2'][1::3].astype(F32)
    W22 = p['W2'][2::3].astype(F32)
    b2 = p['b2'].astype(F32)[None, :]
    v128 = _bcast_spec((1, 128))
    w2_specs = [_bcast_spec((64, 128))] * 3 + [v128]

    feat1, s4, q4 = pl.pallas_call(
        _feat1_stage2_body, grid=(B,),
        in_specs=[rl_in, _cloud_spec((P, K)), _bcast_spec((3, 64)), v64,
                  v64, v64, _bcast_spec((64, 64)), v64,
                  v64, v64, _bcast_spec((64, 64)), v64,
                  v64, v64] + w2_specs,
        out_specs=[_cloud_spec((P, 64)), _cloud_spec((1, 128)),
                   _cloud_spec((1, 128))],
        out_shape=[jax.ShapeDtypeStruct((B, P, 64), F32),
                   jax.ShapeDtypeStruct((B, 1, 128), F32),
                   jax.ShapeDtypeStruct((B, 1, 128), F32)],
    )(rl3, idx3, W1a, b1a, sc1, sh1, W1b, b1b, sc2, sh2, W1c, b1c,
      sc3, sh3, W20, W21, W22, b2)
    sc4, sh4 = stats(s4, q4, E, p['g2'], p['be2'])

    y3 = pl.pallas_call(
        _feat2_head1_body, grid=(B,),
        in_specs=[rl_in, _cloud_spec((P, K)), _cloud_spec((P, 64))]
        + w2_specs + [v128, v128,
                      _bcast_spec((128, 1024)), _bcast_spec((1, 1024))],
        out_specs=[_cloud_spec((1, 1024))],
        out_shape=[jax.ShapeDtypeStruct((B, 1, 1024), F32)],
    )(rl3, idx3, feat1, W20, W21, W22, b2, sc4, sh4,
      p['Wn1'].astype(F32), p['bn1'].astype(F32)[None, :])[0]

    out = pl.pallas_call(
        _head_body,
        out_shape=jax.ShapeDtypeStruct((B, 40), F32),
    )(y3.reshape(B, 1024),
      p['Wn2'].astype(F32), p['bn2'].astype(F32)[None, :],
      p['gn1'].astype(F32)[None, :], p['ben1'].astype(F32)[None, :],
      p['gn2'].astype(F32)[None, :], p['ben2'].astype(F32)[None, :],
      p['Wn3'].astype(F32), p['bn3'].astype(F32)[None, :],
      p['gn3'].astype(F32)[None, :], p['ben3'].astype(F32)[None, :],
      p['Wn4'].astype(F32), p['bn4'].astype(F32)[None, :])
    return out


# speedup vs baseline: 1.8152x; 1.0266x over previous
import jax
import jax.numpy as jnp
from jax import lax
from jax.experimental import pallas as pl

P = 100
K = 20
F32 = jnp.float32


def _iota_j():
    return lax.broadcasted_iota(jnp.int32, (P, P), 1)


def _mm(a, w):
    return jnp.dot(a, w, preferred_element_type=F32)


def _gmm(a, w):
    return _gmm3(a, _gsplit(w))


def _gsplit(w):
    hi = _bz(w)
    r1 = w - hi
    mid = _bz(r1)
    return hi, mid, r1 - mid


def _gmm3(a, parts):
    return (_mm(a, parts[0]) + _mm(a, parts[1])) + _mm(a, parts[2])


def _bz(t):
    return t.astype(jnp.bfloat16).astype(F32)


def _knn_cov_body(pos_ref, idx_ref, cov_ref):
    pos = pos_ref[0]
    dot = jnp.dot(pos, pos.T, preferred_element_type=F32)
    ii = lax.broadcasted_iota(jnp.int32, (P, P), 0)
    jj = _iota_j()
    eye = (ii == jj).astype(F32)
    sq_col = jnp.sum(pos * pos, axis=1, keepdims=True)
    sq_row = jnp.sum(eye * sq_col, axis=0, keepdims=True)
    d = (sq_col + sq_row) - 2.0 * dot + eye * 1e10

    idxm = jnp.zeros((P, K), jnp.int32)
    kk = lax.broadcasted_iota(jnp.int32, (P, K), 1)
    posp = _gsplit(pos)
    cab = [[None] * 3 for _ in range(3)]
    for k in range(K):
        m = jnp.min(d, axis=1, keepdims=True)
        amin = jnp.min(jnp.where(d == m, jj, P), axis=1, keepdims=True)
        oh = (jj == amin)
        idxm = jnp.where(kk == k, amin, idxm)
        ps = _gmm3(oh.astype(F32), posp)
        r = [ps[:, c:c + 1] - pos[:, c:c + 1] for c in range(3)]
        for a in range(3):
            for b in range(a, 3):
                t = r[a] * r[b]
                cab[a][b] = t if cab[a][b] is None else cab[a][b] + t
        d = jnp.where(oh, 1e30, d)

    idx_ref[0] = idxm
    c9 = lax.broadcasted_iota(jnp.int32, (P, 9), 1)
    cov = jnp.zeros((P, 9), F32)
    for a in range(3):
        for b in range(3):
            v = cab[a][b] if b >= a else cab[b][a]
            cov = jnp.where(c9 == 3 * a + b, v / K, cov)
    cov_ref[0] = cov


def _rl_cols(rlm, k):
    return [rlm[:, 3 * k + c:3 * k + c + 1] for c in range(3)]


def _z1(rl, W1a, b1a):
    z = (_bz(rl[0]) * _bz(W1a[0:1, :]) + _bz(rl[1]) * _bz(W1a[1:2, :])
         + _bz(rl[2]) * _bz(W1a[2:3, :])) + b1a
    return jnp.maximum(z, 0.0)


def _stats1_body(rl_ref, W1a_ref, b1a_ref, s_ref, q_ref):
    rlm = rl_ref[0]
    W1a, b1a = W1a_ref[...], b1a_ref[...]
    s = jnp.zeros((1, 64), F32)
    q = jnp.zeros((1, 64), F32)
    for k in range(K):
        a1 = _z1(_rl_cols(rlm, k), W1a, b1a)
        s = s + jnp.sum(a1, axis=0, keepdims=True)
        q = q + jnp.sum(a1 * a1, axis=0, keepdims=True)
    s_ref[0] = s
    q_ref[0] = q


def _stats2_body(rl_ref, W1a_ref, b1a_ref, sc1_ref, sh1_ref, W1b_ref, b1b_ref,
                 s_ref, q_ref):
    rlm = rl_ref[0]
    W1a, b1a = W1a_ref[...], b1a_ref[...]
    sc1, sh1 = sc1_ref[...], sh1_ref[...]
    W1b, b1b = W1b_ref[...], b1b_ref[...]
    s = jnp.zeros((1, 64), F32)
    q = jnp.zeros((1, 64), F32)
    for k in range(K):
        a1 = _z1(_rl_cols(rlm, k), W1a, b1a)
        a2 = jnp.maximum(_mm(a1 * sc1 + sh1, W1b) + b1b, 0.0)
        s = s + jnp.sum(a2, axis=0, keepdims=True)
        q = q + jnp.sum(a2 * a2, axis=0, keepdims=True)
    s_ref[0] = s
    q_ref[0] = q


def _stats3_body(rl_ref, W1a_ref, b1a_ref, sc1_ref, sh1_ref, W1b_ref, b1b_ref,
                 sc2_ref, sh2_ref, W1c_ref, b1c_ref, s_ref, q_ref):
    rlm = rl_ref[0]
    W1a, b1a = W1a_ref[...], b1a_ref[...]
    sc1, sh1 = sc1_ref[...], sh1_ref[...]
    W1b, b1b = W1b_ref[...], b1b_ref[...]
    sc2, sh2 = sc2_ref[...], sh2_ref[...]
    W1c, b1c = W1c_ref[...], b1c_ref[...]
    s = jnp.zeros((1, 64), F32)
    q = jnp.zeros((1, 64), F32)
    for k in range(K):
        a1 = _z1(_rl_cols(rlm, k), W1a, b1a)
        a2 = jnp.maximum(_mm(a1 * sc1 + sh1, W1b) + b1b, 0.0)
        a3 = jnp.maximum(_mm(a2 * sc2 + sh2, W1c) + b1c, 0.0)
        s = s + jnp.sum(a3, axis=0, keepdims=True)
        q = q + jnp.sum(a3 * a3, axis=0, keepdims=True)
    s_ref[0] = s
    q_ref[0] = q


def _stage2(rl, oh, f1p, W20, W21, W22, b2):
    fs = _gmm3(oh, f1p)
    return (_mm(fs * rl[0], W20) + _mm(fs * rl[1], W21)
            + _mm(fs * rl[2], W22)) + b2


def _feat1_stage2_body(rl_ref, idx_ref, W1a_ref, b1a_ref, sc1_ref, sh1_ref,
                       W1b_ref, b1b_ref, sc2_ref, sh2_ref, W1c_ref, b1c_ref,
                       sc3_ref, sh3_ref, W20_ref, W21_ref, W22_ref, b2_ref,
                       feat1_ref, s_ref, q_ref):
    rlm, idxm = rl_ref[0], idx_ref[0]
    W1a, b1a = W1a_ref[...], b1a_ref[...]
    sc1, sh1 = sc1_ref[...], sh1_ref[...]
    W1b, b1b = W1b_ref[...], b1b_ref[...]
    sc2, sh2 = sc2_ref[...], sh2_ref[...]
    W1c, b1c = W1c_ref[...], b1c_ref[...]
    sc3, sh3 = sc3_ref[...], sh3_ref[...]
    W20, W21, W22, b2 = W20_ref[...], W21_ref[...], W22_ref[...], b2_ref[...]

    feat1 = None
    for k in range(K):
        a1 = _z1(_rl_cols(rlm, k), W1a, b1a)
        a2 = jnp.maximum(_mm(a1 * sc1 + sh1, W1b) + b1b, 0.0)
        a3 = jnp.maximum(_mm(a2 * sc2 + sh2, W1c) + b1c, 0.0)
        h = a3 * sc3 + sh3
        feat1 = h if feat1 is None else jnp.maximum(feat1, h)
    feat1_ref[0] = feat1

    jj = _iota_j()
    f1p = _gsplit(feat1)
    s = jnp.zeros((1, 128), F32)
    q = jnp.zeros((1, 128), F32)
    for k in range(K):
        oh = (jj == idxm[:, k:k + 1]).astype(F32)
        a = jnp.maximum(
            _stage2(_rl_cols(rlm, k), oh, f1p, W20, W21, W22, b2), 0.0)
        s = s + jnp.sum(a, axis=0, keepdims=True)
        q = q + jnp.sum(a * a, axis=0, keepdims=True)
    s_ref[0] = s
    q_ref[0] = q


def _feat2_head1_body(rl_ref, idx_ref, feat1_ref, W20_ref, W21_ref, W22_ref,
                      b2_ref, sc4_ref, sh4_ref, Wn1_ref, bn1_ref, y_ref):
    rlm, idxm = rl_ref[0], idx_ref[0]
    feat1 = feat1_ref[0]
    W20, W21, W22, b2 = W20_ref[...], W21_ref[...], W22_ref[...], b2_ref[...]
    sc4, sh4 = sc4_ref[...], sh4_ref[...]
    Wn1, bn1 = Wn1_ref[...], bn1_ref[...]

    jj = _iota_j()
    f1p = _gsplit(feat1)
    feat2 = None
    for k in range(K):
        oh = (jj == idxm[:, k:k + 1]).astype(F32)
        z2 = _stage2(_rl_cols(rlm, k), oh, f1p, W20, W21, W22, b2)
        h2 = jnp.maximum(z2, 0.0) * sc4 + sh4
        feat2 = h2 if feat2 is None else jnp.maximum(feat2, h2)
    y1 = _mm(feat2, Wn1) + bn1
    y_ref[0] = jnp.max(y1, axis=0, keepdims=True)


def _head_body(y_ref, Wn2_ref, bn2_ref, g1_ref, be1_ref, g2_ref, be2_ref,
               Wn3_ref, bn3_ref, g3_ref, be3_ref, Wn4_ref, bn4_ref, out_ref):
    def bn(x, g, be):
        m = jnp.mean(x, axis=0, keepdims=True)
        v = jnp.mean((x - m) * (x - m), axis=0, keepdims=True)
        return (x - m) / jnp.sqrt(v + 1e-5) * g + be

    y = jnp.maximum(y_ref[...], 0.0)
    y = bn(y, g1_ref[...], be1_ref[...])
    y = jnp.maximum(_mm(y, Wn2_ref[...]) + bn2_ref[...], 0.0)
    y = bn(y, g2_ref[...], be2_ref[...])
    y = jnp.maximum(_mm(y, Wn3_ref[...]) + bn3_ref[...], 0.0)
    y = bn(y, g3_ref[...], be3_ref[...])
    z = _mm(y, Wn4_ref[...]) + bn4_ref[...]
    zm = jnp.max(z, axis=1, keepdims=True)
    u = z - zm
    out_ref[...] = u - jnp.log(jnp.sum(jnp.exp(u), axis=1, keepdims=True))


def _cloud_spec(t):
    return pl.BlockSpec((1,) + t, lambda i: (i, 0, 0))


def _bcast_spec(shape):
    n = len(shape)
    return pl.BlockSpec(shape, lambda i: (0,) * n)


def kernel(pos, edge_index, batch, params):
    del edge_index, batch
    p = params
    N = pos.shape[0]
    B = N // P
    E = N * K
    pos = pos.astype(F32)
    pos3 = pos.reshape(B, P, 3)

    idx3, cov3 = pl.pallas_call(
        _knn_cov_body,
        grid=(B,),
        in_specs=[_cloud_spec((P, 3))],
        out_specs=[_cloud_spec((P, K)), _cloud_spec((P, 9))],
        out_shape=[jax.ShapeDtypeStruct((B, P, K), jnp.int32),
                   jax.ShapeDtypeStruct((B, P, 9), F32)],
    )(pos3)

    cov = cov3.reshape(N, 3, 3)
    _, V = jnp.linalg.eigh(cov)
    V_t = jnp.swapaxes(V, -1, -2)
    src = (idx3 + (jnp.arange(B) * P)[:, None, None]).reshape(-1)
    dst = jnp.repeat(jnp.arange(N), K)
    rel = pos[src] - pos[dst]
    rl = jnp.einsum('eij,ej->ei', V_t[dst], rel)
    rl3 = rl.reshape(B, P, K * 3)

    def stats(s, q, n, g, be):
        m = jnp.sum(s, axis=(0, 1)) / n
        v = jnp.sum(q, axis=(0, 1)) / n - m * m
        sc = (g / jnp.sqrt(v + 1e-5))[None, :]
        sh = (be - m * sc[0])[None, :]
        return sc.astype(F32), sh.astype(F32)

    W1a = p['W1a'].astype(F32)
    b1a = p['b1a'].astype(F32)[None, :]
    W1b = p['W1b'].astype(F32)
    b1b = p['b1b'].astype(F32)[None, :]
    W1c = p['W1c'].astype(F32)
    b1c = p['b1c'].astype(F32)[None, :]
    rl_in = _cloud_spec((P, K * 3))
    stat_outs = [_cloud_spec((1, 64)), _cloud_spec((1, 64))]
    stat_shapes = [jax.ShapeDtypeStruct((B, 1, 64), F32)] * 2
    v64 = _bcast_spec((1, 64))

    s1, q1 = pl.pallas_call(
        _stats1_body, grid=(B,),
        in_specs=[rl_in, _bcast_spec((3, 64)), v64],
        out_specs=stat_outs, out_shape=stat_shapes,
    )(rl3, W1a, b1a)
    sc1, sh1 = stats(s1, q1, E, p['g1a'], p['be1a'])

    s2, q2 = pl.pallas_call(
        _stats2_body, grid=(B,),
        in_specs=[rl_in, _bcast_spec((3, 64)), v64, v64, v64,
                  _bcast_spec((64, 64)), v64],
        out_specs=stat_outs, out_shape=stat_shapes,
    )(rl3, W1a, b1a, sc1, sh1, W1b, b1b)
    sc2, sh2 = stats(s2, q2, E, p['g1b'], p['be1b'])

    s3, q3 = pl.pallas_call(
        _stats3_body, grid=(B,),
        in_specs=[rl_in, _bcast_spec((3, 64)), v64, v64, v64,
                  _bcast_spec((64, 64)), v64, v64, v64,
                  _bcast_spec((64, 64)), v64],
        out_specs=stat_outs, out_shape=stat_shapes,
    )(rl3, W1a, b1a, sc1, sh1, W1b, b1b, sc2, sh2, W1c, b1c)
    sc3, sh3 = stats(s3, q3, E, p['g1c'], p['be1c'])

    W20 = p['W2'][0::3].astype(F32)
    W21 = p['W2'][1::3].astype(F32)
    W22 = p['W2'][2::3].astype(F32)
    b2 = p['b2'].astype(F32)[None, :]
    v128 = _bcast_spec((1, 128))
    w2_specs = [_bcast_spec((64, 128))] * 3 + [v128]

    feat1, s4, q4 = pl.pallas_call(
        _feat1_stage2_body, grid=(B,),
        in_specs=[rl_in, _cloud_spec((P, K)), _bcast_spec((3, 64)), v64,
                  v64, v64, _bcast_spec((64, 64)), v64,
                  v64, v64, _bcast_spec((64, 64)), v64,
                  v64, v64] + w2_specs,
        out_specs=[_cloud_spec((P, 64)), _cloud_spec((1, 128)),
                   _cloud_spec((1, 128))],
        out_shape=[jax.ShapeDtypeStruct((B, P, 64), F32),
                   jax.ShapeDtypeStruct((B, 1, 128), F32),
                   jax.ShapeDtypeStruct((B, 1, 128), F32)],
    )(rl3, idx3, W1a, b1a, sc1, sh1, W1b, b1b, sc2, sh2, W1c, b1c,
      sc3, sh3, W20, W21, W22, b2)
    sc4, sh4 = stats(s4, q4, E, p['g2'], p['be2'])

    y3 = pl.pallas_call(
        _feat2_head1_body, grid=(B,),
        in_specs=[rl_in, _cloud_spec((P, K)), _cloud_spec((P, 64))]
        + w2_specs + [v128, v128,
                      _bcast_spec((128, 1024)), _bcast_spec((1, 1024))],
        out_specs=[_cloud_spec((1, 1024))],
        out_shape=[jax.ShapeDtypeStruct((B, 1, 1024), F32)],
    )(rl3, idx3, feat1, W20, W21, W22, b2, sc4, sh4,
      p['Wn1'].astype(F32), p['bn1'].astype(F32)[None, :])[0]

    out = pl.pallas_call(
        _head_body,
        out_shape=jax.ShapeDtypeStruct((B, 40), F32),
    )(y3.reshape(B, 1024),
      p['Wn2'].astype(F32), p['bn2'].astype(F32)[None, :],
      p['gn1'].astype(F32)[None, :], p['ben1'].astype(F32)[None, :],
      p['gn2'].astype(F32)[None, :], p['ben2'].astype(F32)[None, :],
      p['Wn3'].astype(F32), p['bn3'].astype(F32)[None, :],
      p['gn3'].astype(F32)[None, :], p['ben3'].astype(F32)[None, :],
      p['Wn4'].astype(F32), p['bn4'].astype(F32)[None, :])
    return out


# speedup vs baseline: 1.8193x; 1.0023x over previous
import jax
import jax.numpy as jnp
from jax import lax
from jax.experimental import pallas as pl

P = 100
K = 20
F32 = jnp.float32


def _iota_j():
    return lax.broadcasted_iota(jnp.int32, (P, P), 1)


def _mm(a, w):
    return jnp.dot(a, w, preferred_element_type=F32)


def _gmm(a, w):
    return _gmm3(a, _gsplit(w))


def _gsplit(w):
    hi = _bz(w)
    r1 = w - hi
    mid = _bz(r1)
    return hi, mid, r1 - mid


def _gmm3(a, parts):
    return (_mm(a, parts[0]) + _mm(a, parts[1])) + _mm(a, parts[2])


def _bz(t):
    return t.astype(jnp.bfloat16).astype(F32)


def _knn_cov_body(pos_ref, idx_ref, cov_ref):
    pos = pos_ref[0]
    dot = jnp.dot(pos, pos.T, preferred_element_type=F32)
    ii = lax.broadcasted_iota(jnp.int32, (P, P), 0)
    jj = _iota_j()
    eye = (ii == jj).astype(F32)
    sq_col = jnp.sum(pos * pos, axis=1, keepdims=True)
    sq_row = jnp.sum(eye * sq_col, axis=0, keepdims=True)
    d = (sq_col + sq_row) - 2.0 * dot + eye * 1e10

    idxm = jnp.zeros((P, K), jnp.int32)
    kk = lax.broadcasted_iota(jnp.int32, (P, K), 1)
    posp = _gsplit(pos)
    cab = [[None] * 3 for _ in range(3)]
    for k in range(K):
        m = jnp.min(d, axis=1, keepdims=True)
        amin = jnp.min(jnp.where(d == m, jj, P), axis=1, keepdims=True)
        oh = (jj == amin)
        idxm = jnp.where(kk == k, amin, idxm)
        ps = _gmm3(oh.astype(F32), posp)
        r = [ps[:, c:c + 1] - pos[:, c:c + 1] for c in range(3)]
        for a in range(3):
            for b in range(a, 3):
                t = r[a] * r[b]
                cab[a][b] = t if cab[a][b] is None else cab[a][b] + t
        d = jnp.where(oh, 1e30, d)

    idx_ref[0] = idxm
    c9 = lax.broadcasted_iota(jnp.int32, (P, 9), 1)
    cov = jnp.zeros((P, 9), F32)
    for a in range(3):
        for b in range(3):
            v = cab[a][b] if b >= a else cab[b][a]
            cov = jnp.where(c9 == 3 * a + b, v / K, cov)
    cov_ref[0] = cov


def _rl_cols(rlm, k):
    return [rlm[:, 3 * k + c:3 * k + c + 1] for c in range(3)]


def _z1(rl, W1a, b1a):
    z = (_bz(rl[0]) * _bz(W1a[0:1, :]) + _bz(rl[1]) * _bz(W1a[1:2, :])
         + _bz(rl[2]) * _bz(W1a[2:3, :])) + b1a
    return jnp.maximum(z, 0.0)


CG = 25


def _stats1_body(rl_ref, W1a_ref, b1a_ref, s_ref, q_ref):
    rlm = rl_ref[...].reshape(CG * P, K * 3)
    W1a, b1a = W1a_ref[...], b1a_ref[...]
    s = jnp.zeros((1, 64), F32)
    q = jnp.zeros((1, 64), F32)
    for k in range(K):
        a1 = _z1(_rl_cols(rlm, k), W1a, b1a)
        s = s + jnp.sum(a1, axis=0, keepdims=True)
        q = q + jnp.sum(a1 * a1, axis=0, keepdims=True)
    s_ref[0] = s
    q_ref[0] = q


def _stats2_body(rl_ref, W1a_ref, b1a_ref, sc1_ref, sh1_ref, W1b_ref, b1b_ref,
                 s_ref, q_ref):
    rlm = rl_ref[...].reshape(CG * P, K * 3)
    W1a, b1a = W1a_ref[...], b1a_ref[...]
    sc1, sh1 = sc1_ref[...], sh1_ref[...]
    W1b, b1b = W1b_ref[...], b1b_ref[...]
    s = jnp.zeros((1, 64), F32)
    q = jnp.zeros((1, 64), F32)
    for k in range(K):
        a1 = _z1(_rl_cols(rlm, k), W1a, b1a)
        a2 = jnp.maximum(_mm(a1 * sc1 + sh1, W1b) + b1b, 0.0)
        s = s + jnp.sum(a2, axis=0, keepdims=True)
        q = q + jnp.sum(a2 * a2, axis=0, keepdims=True)
    s_ref[0] = s
    q_ref[0] = q


def _stats3_body(rl_ref, W1a_ref, b1a_ref, sc1_ref, sh1_ref, W1b_ref, b1b_ref,
                 sc2_ref, sh2_ref, W1c_ref, b1c_ref, s_ref, q_ref):
    rlm = rl_ref[...].reshape(CG * P, K * 3)
    W1a, b1a = W1a_ref[...], b1a_ref[...]
    sc1, sh1 = sc1_ref[...], sh1_ref[...]
    W1b, b1b = W1b_ref[...], b1b_ref[...]
    sc2, sh2 = sc2_ref[...], sh2_ref[...]
    W1c, b1c = W1c_ref[...], b1c_ref[...]
    s = jnp.zeros((1, 64), F32)
    q = jnp.zeros((1, 64), F32)
    for k in range(K):
        a1 = _z1(_rl_cols(rlm, k), W1a, b1a)
        a2 = jnp.maximum(_mm(a1 * sc1 + sh1, W1b) + b1b, 0.0)
        a3 = jnp.maximum(_mm(a2 * sc2 + sh2, W1c) + b1c, 0.0)
        s = s + jnp.sum(a3, axis=0, keepdims=True)
        q = q + jnp.sum(a3 * a3, axis=0, keepdims=True)
    s_ref[0] = s
    q_ref[0] = q


def _stage2(rl, oh, f1p, W20, W21, W22, b2):
    fs = _gmm3(oh, f1p)
    return (_mm(fs * rl[0], W20) + _mm(fs * rl[1], W21)
            + _mm(fs * rl[2], W22)) + b2


def _feat1_stage2_body(rl_ref, idx_ref, W1a_ref, b1a_ref, sc1_ref, sh1_ref,
                       W1b_ref, b1b_ref, sc2_ref, sh2_ref, W1c_ref, b1c_ref,
                       sc3_ref, sh3_ref, W20_ref, W21_ref, W22_ref, b2_ref,
                       feat1_ref, s_ref, q_ref):
    rlm, idxm = rl_ref[0], idx_ref[0]
    W1a, b1a = W1a_ref[...], b1a_ref[...]
    sc1, sh1 = sc1_ref[...], sh1_ref[...]
    W1b, b1b = W1b_ref[...], b1b_ref[...]
    sc2, sh2 = sc2_ref[...], sh2_ref[...]
    W1c, b1c = W1c_ref[...], b1c_ref[...]
    sc3, sh3 = sc3_ref[...], sh3_ref[...]
    W20, W21, W22, b2 = W20_ref[...], W21_ref[...], W22_ref[...], b2_ref[...]

    feat1 = None
    for k in range(K):
        a1 = _z1(_rl_cols(rlm, k), W1a, b1a)
        a2 = jnp.maximum(_mm(a1 * sc1 + sh1, W1b) + b1b, 0.0)
        a3 = jnp.maximum(_mm(a2 * sc2 + sh2, W1c) + b1c, 0.0)
        h = a3 * sc3 + sh3
        feat1 = h if feat1 is None else jnp.maximum(feat1, h)
    feat1_ref[0] = feat1

    jj = _iota_j()
    f1p = _gsplit(feat1)
    s = jnp.zeros((1, 128), F32)
    q = jnp.zeros((1, 128), F32)
    for k in range(K):
        oh = (jj == idxm[:, k:k + 1]).astype(F32)
        a = jnp.maximum(
            _stage2(_rl_cols(rlm, k), oh, f1p, W20, W21, W22, b2), 0.0)
        s = s + jnp.sum(a, axis=0, keepdims=True)
        q = q + jnp.sum(a * a, axis=0, keepdims=True)
    s_ref[0] = s
    q_ref[0] = q


def _feat2_head1_body(rl_ref, idx_ref, feat1_ref, W20_ref, W21_ref, W22_ref,
                      b2_ref, sc4_ref, sh4_ref, Wn1_ref, bn1_ref, y_ref):
    rlm, idxm = rl_ref[0], idx_ref[0]
    feat1 = feat1_ref[0]
    W20, W21, W22, b2 = W20_ref[...], W21_ref[...], W22_ref[...], b2_ref[...]
    sc4, sh4 = sc4_ref[...], sh4_ref[...]
    Wn1, bn1 = Wn1_ref[...], bn1_ref[...]

    jj = _iota_j()
    f1p = _gsplit(feat1)
    feat2 = None
    for k in range(K):
        oh = (jj == idxm[:, k:k + 1]).astype(F32)
        z2 = _stage2(_rl_cols(rlm, k), oh, f1p, W20, W21, W22, b2)
        h2 = jnp.maximum(z2, 0.0) * sc4 + sh4
        feat2 = h2 if feat2 is None else jnp.maximum(feat2, h2)
    y1 = _mm(feat2, Wn1) + bn1
    y_ref[0] = jnp.max(y1, axis=0, keepdims=True)


def _head_body(y_ref, Wn2_ref, bn2_ref, g1_ref, be1_ref, g2_ref, be2_ref,
               Wn3_ref, bn3_ref, g3_ref, be3_ref, Wn4_ref, bn4_ref, out_ref):
    def bn(x, g, be):
        m = jnp.mean(x, axis=0, keepdims=True)
        v = jnp.mean((x - m) * (x - m), axis=0, keepdims=True)
        return (x - m) / jnp.sqrt(v + 1e-5) * g + be

    y = jnp.maximum(y_ref[...], 0.0)
    y = bn(y, g1_ref[...], be1_ref[...])
    y = jnp.maximum(_mm(y, Wn2_ref[...]) + bn2_ref[...], 0.0)
    y = bn(y, g2_ref[...], be2_ref[...])
    y = jnp.maximum(_mm(y, Wn3_ref[...]) + bn3_ref[...], 0.0)
    y = bn(y, g3_ref[...], be3_ref[...])
    z = _mm(y, Wn4_ref[...]) + bn4_ref[...]
    zm = jnp.max(z, axis=1, keepdims=True)
    u = z - zm
    out_ref[...] = u - jnp.log(jnp.sum(jnp.exp(u), axis=1, keepdims=True))


def _cloud_spec(t):
    return pl.BlockSpec((1,) + t, lambda i: (i, 0, 0))


def _bcast_spec(shape):
    n = len(shape)
    return pl.BlockSpec(shape, lambda i: (0,) * n)


def kernel(pos, edge_index, batch, params):
    del edge_index, batch
    p = params
    N = pos.shape[0]
    B = N // P
    E = N * K
    pos = pos.astype(F32)
    pos3 = pos.reshape(B, P, 3)

    idx3, cov3 = pl.pallas_call(
        _knn_cov_body,
        grid=(B,),
        in_specs=[_cloud_spec((P, 3))],
        out_specs=[_cloud_spec((P, K)), _cloud_spec((P, 9))],
        out_shape=[jax.ShapeDtypeStruct((B, P, K), jnp.int32),
                   jax.ShapeDtypeStruct((B, P, 9), F32)],
    )(pos3)

    cov = cov3.reshape(N, 3, 3)
    _, V = jnp.linalg.eigh(cov)
    V_t = jnp.swapaxes(V, -1, -2)
    src = (idx3 + (jnp.arange(B) * P)[:, None, None]).reshape(-1)
    dst = jnp.repeat(jnp.arange(N), K)
    rel = pos[src] - pos[dst]
    rl = jnp.einsum('eij,ej->ei', V_t[dst], rel)
    rl3 = rl.reshape(B, P, K * 3)

    def stats(s, q, n, g, be):
        m = jnp.sum(s, axis=(0, 1)) / n
        v = jnp.sum(q, axis=(0, 1)) / n - m * m
        sc = (g / jnp.sqrt(v + 1e-5))[None, :]
        sh = (be - m * sc[0])[None, :]
        return sc.astype(F32), sh.astype(F32)

    W1a = p['W1a'].astype(F32)
    b1a = p['b1a'].astype(F32)[None, :]
    W1b = p['W1b'].astype(F32)
    b1b = p['b1b'].astype(F32)[None, :]
    W1c = p['W1c'].astype(F32)
    b1c = p['b1c'].astype(F32)[None, :]
    rl_in = _cloud_spec((P, K * 3))
    rl_chunk = pl.BlockSpec((CG, P, K * 3), lambda i: (i, 0, 0))
    stat_outs = [_cloud_spec((1, 64)), _cloud_spec((1, 64))]
    stat_shapes = [jax.ShapeDtypeStruct((B // CG, 1, 64), F32)] * 2
    v64 = _bcast_spec((1, 64))

    s1, q1 = pl.pallas_call(
        _stats1_body, grid=(B // CG,),
        in_specs=[rl_chunk, _bcast_spec((3, 64)), v64],
        out_specs=stat_outs, out_shape=stat_shapes,
    )(rl3, W1a, b1a)
    sc1, sh1 = stats(s1, q1, E, p['g1a'], p['be1a'])

    s2, q2 = pl.pallas_call(
        _stats2_body, grid=(B // CG,),
        in_specs=[rl_chunk, _bcast_spec((3, 64)), v64, v64, v64,
                  _bcast_spec((64, 64)), v64],
        out_specs=stat_outs, out_shape=stat_shapes,
    )(rl3, W1a, b1a, sc1, sh1, W1b, b1b)
    sc2, sh2 = stats(s2, q2, E, p['g1b'], p['be1b'])

    s3, q3 = pl.pallas_call(
        _stats3_body, grid=(B // CG,),
        in_specs=[rl_chunk, _bcast_spec((3, 64)), v64, v64, v64,
                  _bcast_spec((64, 64)), v64, v64, v64,
                  _bcast_spec((64, 64)), v64],
        out_specs=stat_outs, out_shape=stat_shapes,
    )(rl3, W1a, b1a, sc1, sh1, W1b, b1b, sc2, sh2, W1c, b1c)
    sc3, sh3 = stats(s3, q3, E, p['g1c'], p['be1c'])

    W20 = p['W2'][0::3].astype(F32)
    W21 = p['W2'][1::3].astype(F32)
    W22 = p['W2'][2::3].astype(F32)
    b2 = p['b2'].astype(F32)[None, :]
    v128 = _bcast_spec((1, 128))
    w2_specs = [_bcast_spec((64, 128))] * 3 + [v128]

    feat1, s4, q4 = pl.pallas_call(
        _feat1_stage2_body, grid=(B,),
        in_specs=[rl_in, _cloud_spec((P, K)), _bcast_spec((3, 64)), v64,
                  v64, v64, _bcast_spec((64, 64)), v64,
                  v64, v64, _bcast_spec((64, 64)), v64,
                  v64, v64] + w2_specs,
        out_specs=[_cloud_spec((P, 64)), _cloud_spec((1, 128)),
                   _cloud_spec((1, 128))],
        out_shape=[jax.ShapeDtypeStruct((B, P, 64), F32),
                   jax.ShapeDtypeStruct((B, 1, 128), F32),
                   jax.ShapeDtypeStruct((B, 1, 128), F32)],
    )(rl3, idx3, W1a, b1a, sc1, sh1, W1b, b1b, sc2, sh2, W1c, b1c,
      sc3, sh3, W20, W21, W22, b2)
    sc4, sh4 = stats(s4, q4, E, p['g2'], p['be2'])

    y3 = pl.pallas_call(
        _feat2_head1_body, grid=(B,),
        in_specs=[rl_in, _cloud_spec((P, K)), _cloud_spec((P, 64))]
        + w2_specs + [v128, v128,
                      _bcast_spec((128, 1024)), _bcast_spec((1, 1024))],
        out_specs=[_cloud_spec((1, 1024))],
        out_shape=[jax.ShapeDtypeStruct((B, 1, 1024), F32)],
    )(rl3, idx3, feat1, W20, W21, W22, b2, sc4, sh4,
      p['Wn1'].astype(F32), p['bn1'].astype(F32)[None, :])[0]

    out = pl.pallas_call(
        _head_body,
        out_shape=jax.ShapeDtypeStruct((B, 40), F32),
    )(y3.reshape(B, 1024),
      p['Wn2'].astype(F32), p['bn2'].astype(F32)[None, :],
      p['gn1'].astype(F32)[None, :], p['ben1'].astype(F32)[None, :],
      p['gn2'].astype(F32)[None, :], p['ben2'].astype(F32)[None, :],
      p['Wn3'].astype(F32), p['bn3'].astype(F32)[None, :],
      p['gn3'].astype(F32)[None, :], p['ben3'].astype(F32)[None, :],
      p['Wn4'].astype(F32), p['bn4'].astype(F32)[None, :])
    return out
